# Initial kernel scaffold; baseline (speedup 1.0000x reference)
import jax
import jax.numpy as jnp
from jax.experimental import pallas as pl


def kernel(point_cloud, input):
    raise NotImplementedError("write your pallas kernel here")


# speedup vs baseline: 11.3151x; 11.3151x over previous
import functools

import jax
import jax.numpy as jnp
from jax import lax
from jax.experimental import pallas as pl
from jax.experimental.pallas import tpu as pltpu
from jax.experimental.pallas import tpu_sc as plsc

K = 16
N = 4096
B = 8
M = 256


def _knn_body(pc_ref, q_ref, idx_ref, neg_scr):
    p = pc_ref[0]
    q = q_ref[0]
    r2 = p[0] * p[0] + p[1] * p[1] + p[2] * p[2]
    q2 = q[0] * q[0] + q[1] * q[1] + q[2] * q[2]
    inner = lax.dot_general(
        q.astype(jnp.bfloat16), p.astype(jnp.bfloat16),
        dimension_numbers=(((0,), (0,)), ((), ())),
        preferred_element_type=jnp.float32)
    d2 = (r2[None, :] + q2[:, None]) - 2.0 * inner
    neg_scr[...] = -d2
    iota = lax.broadcasted_iota(jnp.int32, (M, N), 1)
    for t in range(K + 1):
        cur = neg_scr[...]
        mx = jnp.max(cur, axis=1)
        ismax = cur == mx[:, None]
        idxv = jnp.min(jnp.where(ismax, iota, N), axis=1)
        if t > 0:
            idx_ref[0, t - 1, :] = idxv
        if t < K:
            neg_scr[...] = jnp.where(iota == idxv[:, None], float("-inf"), cur)


def _knn_idx(pc):
    return pl.pallas_call(
        _knn_body,
        grid=(B, N // M),
        in_specs=[
            pl.BlockSpec((1, 3, N), lambda b, m: (b, 0, 0)),
            pl.BlockSpec((1, 3, M), lambda b, m: (b, 0, m)),
        ],
        out_specs=pl.BlockSpec((1, K, M), lambda b, m: (b, 0, m)),
        out_shape=jax.ShapeDtypeStruct((B, K, N), jnp.int32),
        scratch_shapes=[pltpu.VMEM((M, N), jnp.float32)],
    )(pc, pc)


def _edge_sc(pc, inp, idx):
    mesh = plsc.VectorSubcoreMesh(core_axis_name="c", subcore_axis_name="s")

    @functools.partial(
        pl.kernel,
        mesh=mesh,
        compiler_params=pltpu.CompilerParams(needs_layout_passes=False),
        out_type=jax.ShapeDtypeStruct((B, 6, K, 3, N), jnp.float32),
        scratch_types=[
            pltpu.VMEM((3 * N,), jnp.float32),
            pltpu.VMEM((3, 3, N), jnp.float32),
            pltpu.VMEM((N,), jnp.int32),
            pltpu.VMEM((3, N), jnp.float32),
            pltpu.VMEM((3, N), jnp.float32),
        ],
    )
    def edge_kernel(pc_hbm, inp_hbm, idx_hbm, out_hbm,
                    pc_v, inp_v, idx_v, nb_v, low_v):
        wid = lax.axis_index("s") * 2 + lax.axis_index("c")
        b = wid // 4
        g = wid % 4
        pltpu.sync_copy(pc_hbm.at[b], pc_v)
        pltpu.sync_copy(inp_hbm.at[b], inp_v)
        for kk in range(4):
            k = g * 4 + kk
            pltpu.sync_copy(idx_hbm.at[b, k], idx_v)

            def gbody(i, _):
                iv = idx_v[pl.ds(i * 16, 16)]
                for c in range(3):
                    nb_v[c, pl.ds(i * 16, 16)] = plsc.load_gather(
                        pc_v, [iv + jnp.int32(c * N)])
                return 0

            lax.fori_loop(0, N // 16, gbody, 0)
            for c in range(3):
                pltpu.sync_copy(inp_v.at[c], out_hbm.at[b, c, k])

                def sbody(i, _):
                    nbs = nb_v[c, pl.ds(i * 16, 16)]
                    for j in range(3):
                        low_v[j, pl.ds(i * 16, 16)] = (
                            nbs - inp_v[c, j, pl.ds(i * 16, 16)])
                    return 0

                lax.fori_loop(0, N // 16, sbody, 0)
                pltpu.sync_copy(low_v, out_hbm.at[b, 3 + c, k])

    return edge_kernel(pc.reshape(B, 3 * N), inp, idx)


def kernel(point_cloud, input):
    idx = _knn_idx(point_cloud)
    edge = _edge_sc(point_cloud, input, idx)
    return edge, idx


# speedup vs baseline: 12.4064x; 1.0964x over previous
import functools

import jax
import jax.numpy as jnp
from jax import lax
from jax.experimental import pallas as pl
from jax.experimental.pallas import tpu as pltpu
from jax.experimental.pallas import tpu_sc as plsc

K = 16
N = 4096
B = 8
M = 256


def _knn_body(pc_ref, q_ref, idx_ref, neg_scr):
    p = pc_ref[0]
    q = q_ref[0]
    r2 = p[0] * p[0] + p[1] * p[1] + p[2] * p[2]
    q2 = q[0] * q[0] + q[1] * q[1] + q[2] * q[2]
    inner = lax.dot_general(
        q.astype(jnp.bfloat16), p.astype(jnp.bfloat16),
        dimension_numbers=(((0,), (0,)), ((), ())),
        preferred_element_type=jnp.float32)
    d2 = (r2[None, :] + q2[:, None]) - 2.0 * inner
    neg_scr[...] = -d2
    iota = lax.broadcasted_iota(jnp.int32, (M, N), 1)
    for t in range(K + 1):
        cur = neg_scr[...]
        idxv = jnp.argmax(cur, axis=1).astype(jnp.int32)
        if t > 0:
            idx_ref[0, t - 1, :] = idxv
        if t < K:
            neg_scr[...] = jnp.where(iota == idxv[:, None], float("-inf"), cur)


def _knn_idx(pc):
    return pl.pallas_call(
        _knn_body,
        grid=(B, N // M),
        in_specs=[
            pl.BlockSpec((1, 3, N), lambda b, m: (b, 0, 0)),
            pl.BlockSpec((1, 3, M), lambda b, m: (b, 0, m)),
        ],
        out_specs=pl.BlockSpec((1, K, M), lambda b, m: (b, 0, m)),
        out_shape=jax.ShapeDtypeStruct((B, K, N), jnp.int32),
        scratch_shapes=[pltpu.VMEM((M, N), jnp.float32)],
    )(pc, pc)


def _edge_sc(pc, inp, idx):
    mesh = plsc.VectorSubcoreMesh(core_axis_name="c", subcore_axis_name="s")

    @functools.partial(
        pl.kernel,
        mesh=mesh,
        compiler_params=pltpu.CompilerParams(needs_layout_passes=False),
        out_type=jax.ShapeDtypeStruct((B, 6, K, 3, N), jnp.float32),
        scratch_types=[
            pltpu.VMEM((3 * N,), jnp.float32),
            pltpu.VMEM((3, 3, N), jnp.float32),
            pltpu.VMEM((N,), jnp.int32),
            pltpu.VMEM((3, N), jnp.float32),
            pltpu.VMEM((3, N), jnp.float32),
        ],
    )
    def edge_kernel(pc_hbm, inp_hbm, idx_hbm, out_hbm,
                    pc_v, inp_v, idx_v, nb_v, low_v):
        wid = lax.axis_index("s") * 2 + lax.axis_index("c")
        b = wid // 4
        g = wid % 4
        pltpu.sync_copy(pc_hbm.at[b], pc_v)
        pltpu.sync_copy(inp_hbm.at[b], inp_v)
        for kk in range(4):
            k = g * 4 + kk
            pltpu.sync_copy(idx_hbm.at[b, k], idx_v)

            def gbody(i, _):
                iv = idx_v[pl.ds(i * 16, 16)]
                for c in range(3):
                    nb_v[c, pl.ds(i * 16, 16)] = plsc.load_gather(
                        pc_v, [iv + jnp.int32(c * N)])
                return 0

            lax.fori_loop(0, N // 16, gbody, 0)
            for c in range(3):
                pltpu.sync_copy(inp_v.at[c], out_hbm.at[b, c, k])

                def sbody(i, _):
                    nbs = nb_v[c, pl.ds(i * 16, 16)]
                    for j in range(3):
                        low_v[j, pl.ds(i * 16, 16)] = (
                            nbs - inp_v[c, j, pl.ds(i * 16, 16)])
                    return 0

                lax.fori_loop(0, N // 16, sbody, 0)
                pltpu.sync_copy(low_v, out_hbm.at[b, 3 + c, k])

    return edge_kernel(pc.reshape(B, 3 * N), inp, idx)


def kernel(point_cloud, input):
    idx = _knn_idx(point_cloud)
    edge = _edge_sc(point_cloud, input, idx)
    return edge, idx
